# baseline (speedup 1.0000x reference)
import functools

import jax
import jax.numpy as jnp
from jax.experimental import pallas as pl

K = 20
A = 21
U = 64
EPS = 1e-06


def _r_kernel(p_ref, q_ref, r_ref):
    p = p_ref[...]
    m = jnp.max(p, axis=1, keepdims=True)
    e = jnp.exp(p - m)
    sm = e / jnp.sum(e, axis=1, keepdims=True)
    q = q_ref[0, :].reshape(1, A, 1)
    r_ref[...] = jnp.log(jnp.maximum(sm / q, EPS))


def _conv_kernel(x_ref, r_ref, z_ref, s_ref, *, pdim):
    x = x_ref[0]
    acc = jnp.zeros((pdim, U), dtype=jnp.float32)
    for k in range(K):
        xs = jax.lax.slice_in_dim(x, k, k + pdim, axis=0)
        acc = acc + jax.lax.dot(xs, r_ref[k],
                                preferred_element_type=jnp.float32)
    z_ref[0] = acc
    s_ref[0, 0] = jnp.max(acc, axis=0)


@jax.jit
def kernel(X, P_logit, Q):
    T, N, F, L, A_ = X.shape
    pdim = L - K + 1
    B = T * N * F

    R = pl.pallas_call(
        _r_kernel,
        out_shape=jax.ShapeDtypeStruct((K, A, U), jnp.float32),
    )(P_logit, Q.reshape(1, A))

    Xf = X.reshape(B, L, A_)
    Z, S = pl.pallas_call(
        functools.partial(_conv_kernel, pdim=pdim),
        grid=(B,),
        in_specs=[
            pl.BlockSpec((1, L, A_), lambda b: (b, 0, 0)),
            pl.BlockSpec((K, A, U), lambda b: (0, 0, 0)),
        ],
        out_specs=[
            pl.BlockSpec((1, pdim, U), lambda b: (b, 0, 0)),
            pl.BlockSpec((1, 1, U), lambda b: (b, 0, 0)),
        ],
        out_shape=[
            jax.ShapeDtypeStruct((B, pdim, U), jnp.float32),
            jax.ShapeDtypeStruct((B, 1, U), jnp.float32),
        ],
    )(Xf, R)

    return (R, S.reshape(T, N, F, U), Z.reshape(T, N, F, pdim, U))


# speedup vs baseline: 1.7054x; 1.7054x over previous
import functools

import jax
import jax.numpy as jnp
from jax.experimental import pallas as pl

K = 20
A = 21
U = 64
EPS = 1e-06
NJ = 6
NH = 4
CHUNK = 84
NB = 4


def _r_kernel(p_ref, q_ref, r_ref, rhs_ref):
    p = p_ref[...]
    m = jnp.max(p, axis=1, keepdims=True)
    e = jnp.exp(p - m)
    sm = e / jnp.sum(e, axis=1, keepdims=True)
    q = q_ref[0, :].reshape(1, A, 1)
    r = jnp.log(jnp.maximum(sm / q, EPS))
    r_ref[...] = r

    rflat = r.reshape(K * A, U).astype(jnp.bfloat16)
    rext = jnp.concatenate(
        [jnp.zeros((64, U), jnp.bfloat16), rflat,
         jnp.zeros((84, U), jnp.bfloat16)], axis=0)
    for j in range(NJ):
        pieces = []
        for h in range(NH):
            off = 64 + CHUNK * j - A * h
            pieces.append(jax.lax.slice_in_dim(rext, off, off + CHUNK, axis=0))
        rhs_ref[j] = jnp.concatenate(pieces, axis=1)


def _conv_kernel(x_ref, rhs_ref, z_ref, s_ref, *, pdim):
    for nb in range(NB):
        v = x_ref[nb].astype(jnp.bfloat16)
        vp = jnp.concatenate([v, jnp.zeros((8, CHUNK), jnp.bfloat16)], axis=0)
        acc = jnp.zeros((128, NH * U), jnp.float32)
        for j in range(NJ):
            lhs = jax.lax.slice_in_dim(vp, j, j + 128, axis=0)
            acc = acc + jax.lax.dot(lhs, rhs_ref[j],
                                    preferred_element_type=jnp.float32)
        smax = None
        for h in range(NH):
            nrows = (pdim - h + NH - 1) // NH
            col = jax.lax.slice_in_dim(acc, h * U, (h + 1) * U, axis=1)
            colv = jax.lax.slice_in_dim(col, 0, nrows, axis=0)
            z_ref[nb, h:pdim:NH, :] = colv
            hm = jnp.max(colv, axis=0)
            smax = hm if smax is None else jnp.maximum(smax, hm)
        s_ref[nb, 0] = smax


@jax.jit
def kernel(X, P_logit, Q):
    T, N, F, L, A_ = X.shape
    pdim = L - K + 1
    B = T * N * F

    R, RHS = pl.pallas_call(
        _r_kernel,
        out_shape=[
            jax.ShapeDtypeStruct((K, A, U), jnp.float32),
            jax.ShapeDtypeStruct((NJ, CHUNK, NH * U), jnp.bfloat16),
        ],
    )(P_logit, Q.reshape(1, A))

    Xv = X.reshape(B, L * A_ // CHUNK, CHUNK)
    Z, S = pl.pallas_call(
        functools.partial(_conv_kernel, pdim=pdim),
        grid=(B // NB,),
        in_specs=[
            pl.BlockSpec((NB, L * A_ // CHUNK, CHUNK), lambda b: (b, 0, 0)),
            pl.BlockSpec((NJ, CHUNK, NH * U), lambda b: (0, 0, 0)),
        ],
        out_specs=[
            pl.BlockSpec((NB, pdim, U), lambda b: (b, 0, 0)),
            pl.BlockSpec((NB, 1, U), lambda b: (b, 0, 0)),
        ],
        out_shape=[
            jax.ShapeDtypeStruct((B, pdim, U), jnp.float32),
            jax.ShapeDtypeStruct((B, 1, U), jnp.float32),
        ],
    )(Xv, RHS)

    return (R, S.reshape(T, N, F, U), Z.reshape(T, N, F, pdim, U))


# speedup vs baseline: 1.9052x; 1.1171x over previous
import functools

import jax
import jax.numpy as jnp
from jax.experimental import pallas as pl

K = 20
A = 21
U = 64
EPS = 1e-06
NJ = 6
NH = 4
CHUNK = 84
NB = 6


def _r_kernel(p_ref, q_ref, r_ref, rhs_ref):
    p = p_ref[...]
    m = jnp.max(p, axis=1, keepdims=True)
    e = jnp.exp(p - m)
    sm = e / jnp.sum(e, axis=1, keepdims=True)
    q = q_ref[0, :].reshape(1, A, 1)
    r = jnp.log(jnp.maximum(sm / q, EPS))
    r_ref[...] = r

    rflat = r.reshape(K * A, U).astype(jnp.bfloat16)
    rext = jnp.concatenate(
        [jnp.zeros((64, U), jnp.bfloat16), rflat,
         jnp.zeros((84, U), jnp.bfloat16)], axis=0)
    for j in range(NJ):
        pieces = []
        for h in range(NH):
            off = 64 + CHUNK * j - A * h
            pieces.append(jax.lax.slice_in_dim(rext, off, off + CHUNK, axis=0))
        rhs_ref[j] = jnp.concatenate(pieces, axis=1)


def _conv_kernel(x_ref, rhs_ref, z_ref, s_ref, *, pdim):
    for nb in range(NB):
        v = x_ref[nb]
        vp = jnp.concatenate([v, jnp.zeros((8, CHUNK), jnp.bfloat16)], axis=0)
        acc = jnp.zeros((128, NH * U), jnp.float32)
        for j in range(NJ):
            lhs = jax.lax.slice_in_dim(vp, j, j + 128, axis=0)
            acc = acc + jax.lax.dot(lhs, rhs_ref[j],
                                    preferred_element_type=jnp.float32)
        smax = None
        for h in range(NH):
            nrows = (pdim - h + NH - 1) // NH
            col = jax.lax.slice_in_dim(acc, h * U, (h + 1) * U, axis=1)
            colv = jax.lax.slice_in_dim(col, 0, nrows, axis=0)
            z_ref[0, 0, nb, h:pdim:NH, :] = colv
            hm = jnp.max(colv, axis=0)
            smax = hm if smax is None else jnp.maximum(smax, hm)
        s_ref[0, 0, nb] = smax


@jax.jit
def kernel(X, P_logit, Q):
    T, N, F, L, A_ = X.shape
    pdim = L - K + 1
    B = T * N * F

    R, RHS = pl.pallas_call(
        _r_kernel,
        out_shape=[
            jax.ShapeDtypeStruct((K, A, U), jnp.float32),
            jax.ShapeDtypeStruct((NJ, CHUNK, NH * U), jnp.bfloat16),
        ],
    )(P_logit, Q.reshape(1, A))

    Xv = X.astype(jnp.bfloat16).reshape(B, L * A_ // CHUNK, CHUNK)
    Z, S = pl.pallas_call(
        functools.partial(_conv_kernel, pdim=pdim),
        grid=(B // NB,),
        in_specs=[
            pl.BlockSpec((NB, L * A_ // CHUNK, CHUNK), lambda b: (b, 0, 0)),
            pl.BlockSpec((NJ, CHUNK, NH * U), lambda b: (0, 0, 0)),
        ],
        out_specs=[
            pl.BlockSpec((1, 1, F, pdim, U), lambda b: (b // N, b % N, 0, 0, 0)),
            pl.BlockSpec((1, 1, F, U), lambda b: (b // N, b % N, 0, 0)),
        ],
        out_shape=[
            jax.ShapeDtypeStruct((T, N, F, pdim, U), jnp.float32),
            jax.ShapeDtypeStruct((T, N, F, U), jnp.float32),
        ],
    )(Xv, RHS)

    return (R, S, Z)


# speedup vs baseline: 1.9958x; 1.0475x over previous
import functools

import jax
import jax.numpy as jnp
from jax.experimental import pallas as pl
from jax.experimental.pallas import tpu as pltpu

K = 20
A = 21
U = 64
EPS = 1e-06
AP = 32
GK = 8
NG = 3
NB = 6


def _r_kernel(p_ref, q_ref, r_ref, rt_ref):
    p = p_ref[...]
    m = jnp.max(p, axis=1, keepdims=True)
    e = jnp.exp(p - m)
    sm = e / jnp.sum(e, axis=1, keepdims=True)
    q = q_ref[0, :].reshape(1, A, 1)
    r = jnp.log(jnp.maximum(sm / q, EPS))
    r_ref[...] = r

    rb = r.astype(jnp.bfloat16)
    zcol = jnp.zeros((U, AP - A), jnp.bfloat16)
    for g in range(NG):
        nk = min(GK, K - GK * g)
        pieces = []
        for i in range(nk):
            rkt = rb[GK * g + i].T
            pieces.append(jnp.concatenate([rkt, zcol], axis=1))
        if nk < GK:
            pieces.append(jnp.zeros((U, AP * (GK - nk)), jnp.bfloat16))
        rt_ref[g] = jnp.concatenate(pieces, axis=1)


def _conv_kernel(x_ref, rt_ref, zt_ref, s_ref, *, pdim, lpad):
    zrow = jnp.zeros((AP - A, lpad), jnp.bfloat16)
    for nb in range(NB):
        xt = x_ref[0, 0, nb].astype(jnp.bfloat16).T
        acc = jnp.zeros((U, lpad), jnp.float32)
        for g in range(NG):
            nk = min(GK, K - GK * g)
            pieces = []
            for i in range(nk):
                k = GK * g + i
                rolled = pltpu.roll(xt, lpad - k, axis=1) if k else xt
                pieces.append(jnp.concatenate([rolled, zrow], axis=0))
            rhs = jnp.concatenate(pieces, axis=0)
            lhs = rt_ref[g]
            if nk < GK:
                lhs = jax.lax.slice_in_dim(lhs, 0, AP * nk, axis=1)
            acc = acc + jax.lax.dot(lhs, rhs,
                                    preferred_element_type=jnp.float32)
        zv = jax.lax.slice_in_dim(acc, 0, pdim, axis=1)
        zt_ref[0, 0, nb] = zv
        s_ref[0, 0, nb] = jnp.max(zv, axis=1)


@jax.jit
def kernel(X, P_logit, Q):
    T, N, F, L, A_ = X.shape
    pdim = L - K + 1

    R, RT = pl.pallas_call(
        _r_kernel,
        out_shape=[
            jax.ShapeDtypeStruct((K, A, U), jnp.float32),
            jax.ShapeDtypeStruct((NG, U, AP * GK), jnp.bfloat16),
        ],
    )(P_logit, Q.reshape(1, A))

    Zt, S = pl.pallas_call(
        functools.partial(_conv_kernel, pdim=pdim, lpad=L),
        grid=(T * N,),
        in_specs=[
            pl.BlockSpec((1, 1, F, L, A_), lambda b: (b // N, b % N, 0, 0, 0)),
            pl.BlockSpec((NG, U, AP * GK), lambda b: (0, 0, 0)),
        ],
        out_specs=[
            pl.BlockSpec((1, 1, F, U, pdim), lambda b: (b // N, b % N, 0, 0, 0)),
            pl.BlockSpec((1, 1, F, U), lambda b: (b // N, b % N, 0, 0)),
        ],
        out_shape=[
            jax.ShapeDtypeStruct((T, N, F, U, pdim), jnp.float32),
            jax.ShapeDtypeStruct((T, N, F, U), jnp.float32),
        ],
    )(X, RT)

    return (R, S, jnp.swapaxes(Zt, 3, 4))


# speedup vs baseline: 3.5376x; 1.7726x over previous
import functools

import jax
import jax.numpy as jnp
from jax.experimental import pallas as pl
from jax.experimental.pallas import tpu as pltpu

K = 20
A = 21
U = 64
EPS = 1e-06
AP = 32
GK = 8
NG = 3


def _r_kernel(p_ref, q_ref, r_ref, rt_ref):
    p = p_ref[...]
    m = jnp.max(p, axis=1, keepdims=True)
    e = jnp.exp(p - m)
    sm = e / jnp.sum(e, axis=1, keepdims=True)
    q = q_ref[0, :].reshape(1, A, 1)
    r = jnp.log(jnp.maximum(sm / q, EPS))
    r_ref[...] = r

    rb = r.astype(jnp.bfloat16)
    zcol = jnp.zeros((U, AP - A), jnp.bfloat16)
    for g in range(NG):
        nk = min(GK, K - GK * g)
        pieces = []
        for i in range(nk):
            rkt = rb[GK * g + i].T
            pieces.append(jnp.concatenate([rkt, zcol], axis=1))
        if nk < GK:
            pieces.append(jnp.zeros((U, AP * (GK - nk)), jnp.bfloat16))
        rt_ref[g] = jnp.concatenate(pieces, axis=1)


def _conv_kernel(x_ref, rt_ref, zt_ref, s_ref, *, pdim, n_batch, lanes):
    xt = x_ref[0].astype(jnp.bfloat16)
    zrow = jnp.zeros((AP - A, lanes), jnp.bfloat16)
    acc = jnp.zeros((U, lanes), jnp.float32)
    for g in range(NG):
        nk = min(GK, K - GK * g)
        pieces = []
        for i in range(nk):
            k = GK * g + i
            rolled = pltpu.roll(xt, lanes - k, axis=1) if k else xt
            pieces.append(jnp.concatenate([rolled, zrow], axis=0))
        rhs = jnp.concatenate(pieces, axis=0)
        lhs = rt_ref[g]
        if nk < GK:
            lhs = jax.lax.slice_in_dim(lhs, 0, AP * nk, axis=1)
        acc = acc + jax.lax.dot(lhs, rhs, preferred_element_type=jnp.float32)
    seg = lanes // n_batch
    for n in range(n_batch):
        blk = jax.lax.slice_in_dim(acc, seg * n, seg * n + pdim, axis=1)
        zt_ref[0, n, 0] = blk
        s_ref[0, 0, n] = jnp.max(blk, axis=1)


@jax.jit
def kernel(X, P_logit, Q):
    T, N, F, L, A_ = X.shape
    pdim = L - K + 1

    R, RT = pl.pallas_call(
        _r_kernel,
        out_shape=[
            jax.ShapeDtypeStruct((K, A, U), jnp.float32),
            jax.ShapeDtypeStruct((NG, U, AP * GK), jnp.bfloat16),
        ],
    )(P_logit, Q.reshape(1, A))

    Xp = X.transpose(0, 2, 4, 1, 3).reshape(T * F, A_, N * L)

    Zt, Sp = pl.pallas_call(
        functools.partial(_conv_kernel, pdim=pdim, n_batch=N, lanes=N * L),
        grid=(T * F,),
        in_specs=[
            pl.BlockSpec((1, A_, N * L), lambda b: (b, 0, 0)),
            pl.BlockSpec((NG, U, AP * GK), lambda b: (0, 0, 0)),
        ],
        out_specs=[
            pl.BlockSpec((1, N, 1, U, pdim), lambda b: (b // F, 0, b % F, 0, 0)),
            pl.BlockSpec((1, 1, N, U), lambda b: (b // F, b % F, 0, 0)),
        ],
        out_shape=[
            jax.ShapeDtypeStruct((T, N, F, U, pdim), jnp.float32),
            jax.ShapeDtypeStruct((T, F, N, U), jnp.float32),
        ],
    )(Xp, RT)

    return (R, Sp.transpose(0, 2, 1, 3), jnp.swapaxes(Zt, 3, 4))
